# baseline (speedup 1.0000x reference)
import functools

import jax
import jax.numpy as jnp
from jax import lax
from jax.experimental import pallas as pl
from jax.experimental.pallas import tpu as pltpu
from jax.experimental.pallas import tpu_sc as plsc

B = 16384
EMB = 16
NC = 2
NS = 16
NW = NC * NS
BPW = B // NW
CH = 128
NCH = BPW // CH
GRP = CH // 16

_mesh = plsc.VectorSubcoreMesh(core_axis_name="c", subcore_axis_name="s")


@functools.partial(
    pl.kernel,
    mesh=_mesh,
    out_type=jax.ShapeDtypeStruct((B,), jnp.float32),
    compiler_params=pltpu.CompilerParams(needs_layout_passes=False),
    scratch_types=[
        pltpu.VMEM((BPW,), jnp.int32),
        pltpu.VMEM((BPW,), jnp.int32),
        pltpu.VMEM((BPW,), jnp.int32),
        pltpu.VMEM((BPW,), jnp.int32),
        pltpu.VMEM((CH, 128), jnp.float32),
        pltpu.VMEM((CH, 128), jnp.float32),
        pltpu.VMEM((BPW,), jnp.float32),
        pltpu.VMEM((32,), jnp.float32),
        pltpu.SemaphoreType.DMA,
        pltpu.SemaphoreType.DMA,
    ],
)
def _sc_fwd(mi_hbm, gi_hbm, m_hbm, g_hbm, wb_hbm, out_hbm,
            midx_v, gidx_v, mgrp_v, ggrp_v, mbuf_v, gbuf_v, out_v, wb_v,
            sem_m, sem_g):
    wid = lax.axis_index("s") * NC + lax.axis_index("c")
    base = wid * BPW

    pltpu.sync_copy(mi_hbm.at[pl.ds(base, BPW)], midx_v)
    pltpu.sync_copy(gi_hbm.at[pl.ds(base, BPW)], gidx_v)
    pltpu.sync_copy(wb_hbm, wb_v)

    for r in range(BPW // 16):
        s = pl.ds(r * 16, 16)
        mgrp_v[s] = lax.shift_right_logical(midx_v[s], 3)
        ggrp_v[s] = lax.shift_right_logical(gidx_v[s], 3)

    lane = lax.iota(jnp.int32, 16)
    wv = wb_v[pl.ds(0, 16)]
    bv = wb_v[pl.ds(16, 16)]
    for j in range(NCH):
        cp_m = pltpu.async_copy(
            m_hbm.at[mgrp_v.at[pl.ds(j * CH, CH)]], mbuf_v, sem_m)
        cp_g = pltpu.async_copy(
            g_hbm.at[ggrp_v.at[pl.ds(j * CH, CH)]], gbuf_v, sem_g)
        cp_m.wait()
        cp_g.wait()
        for k in range(GRP):
            s = pl.ds(j * CH + k * 16, 16)
            rowv = k * 16 + lane
            mcol = (midx_v[s] & 7) * EMB
            gcol = (gidx_v[s] & 7) * EMB
            acc = jnp.zeros((16,), jnp.float32)
            for c in range(EMB):
                mv = plsc.load_gather(mbuf_v, [rowv, mcol + c])
                gv = plsc.load_gather(gbuf_v, [rowv, gcol + c])
                acc = acc + mv * gv
            t = acc * wv + bv
            y = 1.0 / (1.0 + jnp.exp(-t))
            out_v[pl.ds(j * CH + k * 16, 16)] = y

    pltpu.sync_copy(out_v, out_hbm.at[pl.ds(base, BPW)])


def kernel(x, m_table, g_table, W, b):
    mi = x[:, 0]
    gi = x[:, 1]
    m128 = m_table.reshape(-1, 128)
    g128 = g_table.reshape(-1, 128)
    wb = jnp.concatenate([jnp.full((16,), W[0, 0], jnp.float32),
                          jnp.full((16,), b[0], jnp.float32)])
    out = _sc_fwd(mi, gi, m128, g128, wb)
    return out.reshape(B, 1)


# speedup vs baseline: 1.5975x; 1.5975x over previous
import functools

import jax
import jax.numpy as jnp
from jax import lax
from jax.experimental import pallas as pl
from jax.experimental.pallas import tpu as pltpu
from jax.experimental.pallas import tpu_sc as plsc

B = 16384
EMB = 16
NC = 2
NS = 16
NW = NC * NS
BPW = B // NW
PASS = 256
NP = BPW // PASS
NGP = PASS // 16

_mesh = plsc.VectorSubcoreMesh(core_axis_name="c", subcore_axis_name="s")


@functools.partial(
    pl.kernel,
    mesh=_mesh,
    out_type=jax.ShapeDtypeStruct((B,), jnp.float32),
    compiler_params=pltpu.CompilerParams(needs_layout_passes=False),
    scratch_types=[
        pltpu.VMEM((BPW,), jnp.int32),
        pltpu.VMEM((BPW,), jnp.int32),
        pltpu.VMEM((PASS, EMB), jnp.float32),
        pltpu.VMEM((PASS, EMB), jnp.float32),
        pltpu.VMEM((BPW,), jnp.float32),
        pltpu.VMEM((32,), jnp.float32),
        pltpu.SemaphoreType.DMA,
        pltpu.SemaphoreType.DMA,
    ],
)
def _sc_fwd(mi_hbm, gi_hbm, m_hbm, g_hbm, wb_hbm, out_hbm,
            midx_v, gidx_v, mbuf_v, gbuf_v, out_v, wb_v, sem_m, sem_g):
    wid = lax.axis_index("s") * NC + lax.axis_index("c")
    base = wid * BPW

    pltpu.sync_copy(mi_hbm.at[pl.ds(base, BPW)], midx_v)
    pltpu.sync_copy(gi_hbm.at[pl.ds(base, BPW)], gidx_v)
    pltpu.sync_copy(wb_hbm, wb_v)

    lane = lax.iota(jnp.int32, 16)
    wv = wb_v[pl.ds(0, 16)]
    bv = wb_v[pl.ds(16, 16)]

    for p in range(NP):
        poff = p * PASS

        def issue(r, carry):
            mids = midx_v[pl.ds(poff + r * 16, 16)]
            gids = gidx_v[pl.ds(poff + r * 16, 16)]
            for j in range(16):
                slot = r * 16 + j
                pltpu.async_copy(m_hbm.at[mids[j]], mbuf_v.at[slot], sem_m)
                pltpu.async_copy(g_hbm.at[gids[j]], gbuf_v.at[slot], sem_g)
            return carry

        lax.fori_loop(0, NGP, issue, 0)
        pltpu.make_async_copy(m_hbm.at[pl.ds(0, PASS)], mbuf_v, sem_m).wait()
        pltpu.make_async_copy(m_hbm.at[pl.ds(0, PASS)], gbuf_v, sem_g).wait()

        for r in range(NGP):
            rowv = r * 16 + lane
            acc = jnp.zeros((16,), jnp.float32)
            for c in range(EMB):
                cv = jnp.full((16,), c, jnp.int32)
                mv = plsc.load_gather(mbuf_v, [rowv, cv])
                gv = plsc.load_gather(gbuf_v, [rowv, cv])
                acc = acc + mv * gv
            t = acc * wv + bv
            y = 1.0 / (1.0 + jnp.exp(-t))
            out_v[pl.ds(poff + r * 16, 16)] = y

    pltpu.sync_copy(out_v, out_hbm.at[pl.ds(base, BPW)])


def kernel(x, m_table, g_table, W, b):
    mi = x[:, 0]
    gi = x[:, 1]
    wb = jnp.concatenate([jnp.full((16,), W[0, 0], jnp.float32),
                          jnp.full((16,), b[0], jnp.float32)])
    out = _sc_fwd(mi, gi, m_table, g_table, wb)
    return out.reshape(B, 1)
